# Initial kernel scaffold; baseline (speedup 1.0000x reference)
import jax
import jax.numpy as jnp
from jax.experimental import pallas as pl


def kernel(X, edge_index, MLPX_W, MLPX_b, W_adj, W_W, W_b, out_W, out_b):
    raise NotImplementedError("write your pallas kernel here")


# speedup vs baseline: 4.8420x; 4.8420x over previous
import dataclasses

import jax
import jax.numpy as jnp
from jax import lax
from jax.experimental import pallas as pl
from jax.experimental.pallas import tpu as pltpu
from jax.experimental.pallas import tpu_sc as plsc

_N = 10000
_E = 320000
_FEAT = 128
_HID = 256
_HALF = _HID // 2
_K = 128
_NSUB = 16
_ROWS_PER_SUB = 624
_TAIL_BASE = _ROWS_PER_SUB * _NSUB
_TAIL_ROWS = _N - _TAIL_BASE
_NCHUNK = _E // _K
_CHUNK_ITERS = -(-_NCHUNK // _NSUB)
_DROWS = 80
_LANES = 16


def _compiler_params():
    cp = pltpu.CompilerParams()
    if "needs_layout_passes" in pltpu.CompilerParams.__dataclass_fields__:
        cp = dataclasses.replace(cp, needs_layout_passes=False)
    return cp


def _seg_body(w0, w1, row, col, zval, zdeg, ident, s0_out, s1_out, deg_out,
              rowbuf, colbuf, vals, identbuf, degtile, acc, dshared):
    cid = lax.axis_index("c")
    sid = lax.axis_index("s")
    base = sid * _ROWS_PER_SUB
    sl = pl.ds(base, _ROWS_PER_SUB)
    tl = pl.ds(_TAIL_BASE, _TAIL_ROWS)

    pltpu.sync_copy(zval.at[pl.ds(0, _ROWS_PER_SUB)], acc.at[sl])

    @pl.when(sid == _NSUB - 1)
    def _():
        pltpu.sync_copy(zval.at[pl.ds(0, _TAIL_ROWS)], acc.at[tl])

    @pl.when(cid == 0)
    def _():
        pltpu.sync_copy(zdeg, degtile)
        pltpu.sync_copy(ident, identbuf)

        @pl.when(sid == 0)
        def _():
            pltpu.sync_copy(zdeg, dshared)

    plsc.subcore_barrier()

    ones = jnp.full((_LANES,), 1.0, jnp.float32)

    @pl.loop(0, _CHUNK_ITERS)
    def _(i):
        j = i * _NSUB + sid

        @pl.when(j < _NCHUNK)
        def _():
            pltpu.sync_copy(row.at[j], rowbuf)
            pltpu.sync_copy(col.at[j], colbuf)

            @pl.when(cid == 0)
            def _():
                pltpu.sync_copy(w0.at[rowbuf.at[0]], vals)

            @pl.when(cid == 1)
            def _():
                pltpu.sync_copy(w1.at[rowbuf.at[0]], vals)

            pltpu.sync_copy(vals, acc.at[colbuf.at[0]], add=True)

            @pl.when(cid == 0)
            def _():
                for k in range(_K // _LANES):
                    iv = colbuf[0, pl.ds(k * _LANES, _LANES)]
                    r = jnp.right_shift(iv, 7)
                    c = jnp.bitwise_and(iv, 127)
                    plsc.addupdate_scatter(degtile, [r, c], ones)

    plsc.subcore_barrier()

    @pl.when(cid == 0)
    def _():
        pltpu.sync_copy(degtile, dshared.at[identbuf.at[0]], add=True)

    plsc.subcore_barrier()

    @pl.when(cid == 0)
    def _():
        pltpu.sync_copy(acc.at[sl], s0_out.at[sl])

        @pl.when(sid == _NSUB - 1)
        def _():
            pltpu.sync_copy(acc.at[tl], s0_out.at[tl])

        @pl.when(sid < _DROWS // 8)
        def _():
            dsl = pl.ds(sid * 8, 8)
            pltpu.sync_copy(dshared.at[dsl], deg_out.at[dsl])

    @pl.when(cid == 1)
    def _():
        pltpu.sync_copy(acc.at[sl], s1_out.at[sl])

        @pl.when(sid == _NSUB - 1)
        def _():
            pltpu.sync_copy(acc.at[tl], s1_out.at[tl])


def _sc_segment_sum(w0, w1, row, col):
    f32 = jnp.float32
    zval = jnp.zeros((_ROWS_PER_SUB, _HALF), f32)
    zdeg = jnp.zeros((_DROWS, 128), f32)
    ident = jnp.arange(_DROWS, dtype=jnp.int32).reshape(1, _DROWS)
    mesh = plsc.VectorSubcoreMesh(core_axis_name="c", subcore_axis_name="s",
                                  num_cores=2, num_subcores=_NSUB)
    fn = pl.kernel(
        _seg_body,
        out_type=[
            jax.ShapeDtypeStruct((_N, _HALF), f32),
            jax.ShapeDtypeStruct((_N, _HALF), f32),
            jax.ShapeDtypeStruct((_DROWS, 128), f32),
        ],
        mesh=mesh,
        compiler_params=_compiler_params(),
        scratch_types=[
            pltpu.VMEM((1, _K), jnp.int32),
            pltpu.VMEM((1, _K), jnp.int32),
            pltpu.VMEM((_K, _HALF), f32),
            pltpu.VMEM((1, _DROWS), jnp.int32),
            pltpu.VMEM((_DROWS, 128), f32),
            pltpu.VMEM_SHARED((_N, _HALF), f32),
            pltpu.VMEM_SHARED((_DROWS, 128), f32),
        ],
    )
    return fn(w0, w1, row, col, zval, zdeg, ident)


_BN = 1000


def _dense_body(x, s0, s1, dg, w1, b1, ww, wb, ow, ob, o):
    hi = lax.Precision.HIGHEST
    f32 = jnp.float32
    hx = lax.dot_general(x[...], w1[...], (((1,), (1,)), ((), ())),
                         precision=hi, preferred_element_type=f32)
    hx = jnp.maximum(hx + b1[...][None, :], 0.0)
    scale = 1.0 + 1.0 / jnp.maximum(dg[...], 1.0)
    ha0 = jnp.maximum(s0[...] * scale, 0.0)
    ha1 = jnp.maximum(s1[...] * scale, 0.0)
    g = lax.dot_general(ww[...], ow[...], (((0,), (1,)), ((), ())),
                        precision=hi, preferred_element_type=f32)
    cb = lax.dot_general(wb[...][None, :], ow[...], (((1,), (1,)), ((), ())),
                         precision=hi, preferred_element_type=f32)
    cat = jnp.concatenate([hx, ha0, ha1], axis=1)
    y = lax.dot_general(cat, g, (((1,), (0,)), ((), ())),
                        precision=hi, preferred_element_type=f32)
    o[...] = 2.0 * y + 2.0 * cb + ob[...][None, :]


def _dense(X, s0, s1, dg, W1, b1, WW, wb, OW, ob):
    nc = ob.shape[0]
    return pl.pallas_call(
        _dense_body,
        grid=(_N // _BN,),
        in_specs=[
            pl.BlockSpec((_BN, _FEAT), lambda i: (i, 0)),
            pl.BlockSpec((_BN, _HALF), lambda i: (i, 0)),
            pl.BlockSpec((_BN, _HALF), lambda i: (i, 0)),
            pl.BlockSpec((_BN, 1), lambda i: (i, 0)),
            pl.BlockSpec((_HID, _FEAT), lambda i: (0, 0)),
            pl.BlockSpec((_HID,), lambda i: (0,)),
            pl.BlockSpec((_HID, 2 * _HID), lambda i: (0, 0)),
            pl.BlockSpec((_HID,), lambda i: (0,)),
            pl.BlockSpec((nc, _HID), lambda i: (0, 0)),
            pl.BlockSpec((nc,), lambda i: (0,)),
        ],
        out_specs=pl.BlockSpec((_BN, nc), lambda i: (i, 0)),
        out_shape=jax.ShapeDtypeStruct((_N, nc), jnp.float32),
    )(X, s0, s1, dg, W1, b1, WW, wb, OW, ob)


def kernel(X, edge_index, MLPX_W, MLPX_b, W_adj, W_W, W_b, out_W, out_b):
    row = edge_index[0].astype(jnp.int32).reshape(_NCHUNK, 1, _K)
    col = edge_index[1].astype(jnp.int32).reshape(_NCHUNK, 1, _K)
    w0 = W_adj[:, :_HALF]
    w1 = W_adj[:, _HALF:]
    s0, s1, deg80 = _sc_segment_sum(w0, w1, row, col)
    dg = deg80.reshape(_DROWS * 128)[:_N].reshape(_N, 1)
    return _dense(X, s0, s1, dg, MLPX_W, MLPX_b, W_W, W_b, out_W, out_b)


# speedup vs baseline: 7.9641x; 1.6448x over previous
import dataclasses

import jax
import jax.numpy as jnp
from jax import lax
from jax.experimental import pallas as pl
from jax.experimental.pallas import tpu as pltpu
from jax.experimental.pallas import tpu_sc as plsc

_N = 10000
_E = 320000
_FEAT = 128
_HID = 256
_HALF = _HID // 2
_K = 128
_NSUB = 16
_ROWS_PER_SUB = 624
_TAIL_BASE = _ROWS_PER_SUB * _NSUB
_TAIL_ROWS = _N - _TAIL_BASE
_NCHUNK = _E // _K
_CHUNK_ITERS = -(-_NCHUNK // _NSUB)
_NCHUNK_PAD = _CHUNK_ITERS * _NSUB + _NSUB - 4
_NBUF = 2
_GROUPS = -(-_CHUNK_ITERS // _NBUF)
_DROWS = 80
_LANES = 16


def _compiler_params():
    cp = pltpu.CompilerParams()
    if "needs_layout_passes" in pltpu.CompilerParams.__dataclass_fields__:
        cp = dataclasses.replace(cp, needs_layout_passes=False)
    return cp


def _seg_body(w0, w1, rc, zval, zdeg, ident, s0_out, s1_out, deg_out,
              ibuf, vals, identbuf, degtile, acc, dshared,
              semi0, semi1, semg0, semg1):
    cid = lax.axis_index("c")
    sid = lax.axis_index("s")
    base = sid * _ROWS_PER_SUB
    sl = pl.ds(base, _ROWS_PER_SUB)
    tl = pl.ds(_TAIL_BASE, _TAIL_ROWS)
    semi = (semi0, semi1)
    semg = (semg0, semg1)

    start = sid * (_CHUNK_ITERS - 1) + jnp.minimum(sid, 4)
    cnt = jnp.where(sid < 4, _CHUNK_ITERS, _CHUNK_ITERS - 1)

    for b in range(_NBUF):
        pltpu.async_copy(rc.at[start + b], ibuf.at[b], semi[b])
    pltpu.sync_copy(zval.at[pl.ds(0, _ROWS_PER_SUB)], acc.at[sl])

    @pl.when(sid == _NSUB - 1)
    def _():
        pltpu.sync_copy(zval.at[pl.ds(0, _TAIL_ROWS)], acc.at[tl])

    @pl.when(cid == 0)
    def _():
        pltpu.sync_copy(zdeg, degtile)
        pltpu.sync_copy(ident, identbuf)

        @pl.when(sid == 0)
        def _():
            pltpu.sync_copy(zdeg, dshared)

    plsc.subcore_barrier()

    ones = jnp.full((_LANES,), 1.0, jnp.float32)

    @pl.loop(0, _GROUPS)
    def _(p):
        for b in range(_NBUF):
            k = p * _NBUF + b

            @pl.when(k < _CHUNK_ITERS)
            def _(b=b, k=k):
                pltpu.make_async_copy(rc.at[start + k], ibuf.at[b],
                                      semi[b]).wait()

                @pl.when(cid == 0)
                def _():
                    pltpu.async_copy(
                        w0.at[ibuf.at[b, 0]], vals.at[b], semg[b])

                @pl.when(cid == 1)
                def _():
                    pltpu.async_copy(
                        w1.at[ibuf.at[b, 0]], vals.at[b], semg[b])

        for b in range(_NBUF):
            k = p * _NBUF + b

            @pl.when(k < _CHUNK_ITERS)
            def _(b=b, k=k):
                pltpu.make_async_copy(
                    w0.at[ibuf.at[b, 0]], vals.at[b], semg[b]).wait()

            @pl.when(k < cnt)
            def _(b=b, k=k):
                pltpu.sync_copy(vals.at[b], acc.at[ibuf.at[b, 1]],
                                add=True)

                @pl.when(cid == 0)
                def _():
                    for m in range(_K // _LANES):
                        iv = ibuf[b, 1, pl.ds(m * _LANES, _LANES)]
                        r = jnp.right_shift(iv, 7)
                        c = jnp.bitwise_and(iv, 127)
                        plsc.addupdate_scatter(degtile, [r, c], ones)

            @pl.when(k + _NBUF < _CHUNK_ITERS)
            def _(b=b, k=k):
                pltpu.async_copy(rc.at[start + k + _NBUF], ibuf.at[b],
                                 semi[b])

    plsc.subcore_barrier()

    @pl.when(cid == 0)
    def _():
        pltpu.sync_copy(degtile, dshared.at[identbuf.at[0]], add=True)

    plsc.subcore_barrier()

    @pl.when(cid == 0)
    def _():
        pltpu.sync_copy(acc.at[sl], s0_out.at[sl])

        @pl.when(sid == _NSUB - 1)
        def _():
            pltpu.sync_copy(acc.at[tl], s0_out.at[tl])

        @pl.when(sid < _DROWS // 8)
        def _():
            dsl = pl.ds(sid * 8, 8)
            pltpu.sync_copy(dshared.at[dsl], deg_out.at[dsl])

    @pl.when(cid == 1)
    def _():
        pltpu.sync_copy(acc.at[sl], s1_out.at[sl])

        @pl.when(sid == _NSUB - 1)
        def _():
            pltpu.sync_copy(acc.at[tl], s1_out.at[tl])


def _sc_segment_sum(w0, w1, rc):
    f32 = jnp.float32
    zval = jnp.zeros((_ROWS_PER_SUB, _HALF), f32)
    zdeg = jnp.zeros((_DROWS, 128), f32)
    ident = jnp.arange(_DROWS, dtype=jnp.int32).reshape(1, _DROWS)
    mesh = plsc.VectorSubcoreMesh(core_axis_name="c", subcore_axis_name="s",
                                  num_cores=2, num_subcores=_NSUB)
    fn = pl.kernel(
        _seg_body,
        out_type=[
            jax.ShapeDtypeStruct((_N, _HALF), f32),
            jax.ShapeDtypeStruct((_N, _HALF), f32),
            jax.ShapeDtypeStruct((_DROWS, 128), f32),
        ],
        mesh=mesh,
        compiler_params=_compiler_params(),
        scratch_types=[
            pltpu.VMEM((_NBUF, 2, _K), jnp.int32),
            pltpu.VMEM((_NBUF, _K, _HALF), f32),
            pltpu.VMEM((1, _DROWS), jnp.int32),
            pltpu.VMEM((_DROWS, 128), f32),
            pltpu.VMEM_SHARED((_N, _HALF), f32),
            pltpu.VMEM_SHARED((_DROWS, 128), f32),
        ] + [pltpu.SemaphoreType.DMA] * (2 * _NBUF),
    )
    return fn(w0, w1, rc, zval, zdeg, ident)


_BN = 1000


def _dense_body(x, s0, s1, dg, w1, b1, ww, wb, ow, ob, o):
    hi = lax.Precision.HIGHEST
    f32 = jnp.float32
    hx = lax.dot_general(x[...], w1[...], (((1,), (1,)), ((), ())),
                         precision=hi, preferred_element_type=f32)
    hx = jnp.maximum(hx + b1[...][None, :], 0.0)
    scale = 1.0 + 1.0 / jnp.maximum(dg[...], 1.0)
    ha0 = jnp.maximum(s0[...] * scale, 0.0)
    ha1 = jnp.maximum(s1[...] * scale, 0.0)
    g = lax.dot_general(ww[...], ow[...], (((0,), (1,)), ((), ())),
                        precision=hi, preferred_element_type=f32)
    cb = lax.dot_general(wb[...][None, :], ow[...], (((1,), (1,)), ((), ())),
                         precision=hi, preferred_element_type=f32)
    cat = jnp.concatenate([hx, ha0, ha1], axis=1)
    y = lax.dot_general(cat, g, (((1,), (0,)), ((), ())),
                        precision=hi, preferred_element_type=f32)
    o[...] = 2.0 * y + 2.0 * cb + ob[...][None, :]


def _dense(X, s0, s1, dg, W1, b1, WW, wb, OW, ob):
    nc = ob.shape[0]
    return pl.pallas_call(
        _dense_body,
        grid=(_N // _BN,),
        in_specs=[
            pl.BlockSpec((_BN, _FEAT), lambda i: (i, 0)),
            pl.BlockSpec((_BN, _HALF), lambda i: (i, 0)),
            pl.BlockSpec((_BN, _HALF), lambda i: (i, 0)),
            pl.BlockSpec((_BN, 1), lambda i: (i, 0)),
            pl.BlockSpec((_HID, _FEAT), lambda i: (0, 0)),
            pl.BlockSpec((_HID,), lambda i: (0,)),
            pl.BlockSpec((_HID, 2 * _HID), lambda i: (0, 0)),
            pl.BlockSpec((_HID,), lambda i: (0,)),
            pl.BlockSpec((nc, _HID), lambda i: (0, 0)),
            pl.BlockSpec((nc,), lambda i: (0,)),
        ],
        out_specs=pl.BlockSpec((_BN, nc), lambda i: (i, 0)),
        out_shape=jax.ShapeDtypeStruct((_N, nc), jnp.float32),
    )(X, s0, s1, dg, W1, b1, WW, wb, OW, ob)


def kernel(X, edge_index, MLPX_W, MLPX_b, W_adj, W_W, W_b, out_W, out_b):
    pad = jnp.zeros((_NCHUNK_PAD - _NCHUNK, 2, _K), jnp.int32)
    rc = jnp.concatenate(
        [edge_index.astype(jnp.int32).reshape(2, _NCHUNK, _K).transpose(1, 0, 2),
         pad])
    w0 = W_adj[:, :_HALF]
    w1 = W_adj[:, _HALF:]
    s0, s1, deg80 = _sc_segment_sum(w0, w1, rc)
    dg = deg80.reshape(_DROWS * 128)[:_N].reshape(_N, 1)
    return _dense(X, s0, s1, dg, MLPX_W, MLPX_b, W_W, W_b, out_W, out_b)


# speedup vs baseline: 8.2662x; 1.0379x over previous
import dataclasses

import jax
import jax.numpy as jnp
from jax import lax
from jax.experimental import pallas as pl
from jax.experimental.pallas import tpu as pltpu
from jax.experimental.pallas import tpu_sc as plsc

_N = 10000
_E = 320000
_FEAT = 128
_HID = 256
_HALF = _HID // 2
_K = 128
_NSUB = 16
_ROWS_PER_SUB = 624
_TAIL_BASE = _ROWS_PER_SUB * _NSUB
_TAIL_ROWS = _N - _TAIL_BASE
_NCHUNK = _E // _K
_CHUNK_ITERS = -(-_NCHUNK // _NSUB)
_NCHUNK_PAD = _CHUNK_ITERS * _NSUB + _NSUB - 4
_NBUF = 2
_GROUPS = -(-_CHUNK_ITERS // _NBUF)
_DROWS = 80
_LANES = 16


def _compiler_params():
    cp = pltpu.CompilerParams()
    if "needs_layout_passes" in pltpu.CompilerParams.__dataclass_fields__:
        cp = dataclasses.replace(cp, needs_layout_passes=False)
    return cp


def _seg_body(w0, w1, rc, zval, zdeg, ident, s0_out, s1_out, deg_out,
              ibuf, cpriv, vals, identbuf, degtile, acc, dshared,
              semi0, semi1, semg0, semg1, sems0, sems1):
    cid = lax.axis_index("c")
    sid = lax.axis_index("s")
    base = sid * _ROWS_PER_SUB
    sl = pl.ds(base, _ROWS_PER_SUB)
    tl = pl.ds(_TAIL_BASE, _TAIL_ROWS)
    semi = (semi0, semi1)
    semg = (semg0, semg1)
    sems = (sems0, sems1)

    start = sid * (_CHUNK_ITERS - 1) + jnp.minimum(sid, 4)
    cnt = jnp.where(sid < 4, _CHUNK_ITERS, _CHUNK_ITERS - 1)

    for b in range(_NBUF):
        pltpu.async_copy(rc.at[start + b], ibuf.at[b], semi[b])
    pltpu.sync_copy(zval.at[pl.ds(0, _ROWS_PER_SUB)], acc.at[sl])

    @pl.when(sid == _NSUB - 1)
    def _():
        pltpu.sync_copy(zval.at[pl.ds(0, _TAIL_ROWS)], acc.at[tl])

    @pl.when(cid == 0)
    def _():
        pltpu.sync_copy(zdeg, degtile)
        pltpu.sync_copy(ident, identbuf)

        @pl.when(sid == 0)
        def _():
            pltpu.sync_copy(zdeg, dshared)

    plsc.subcore_barrier()

    ones = jnp.full((_LANES,), 1.0, jnp.float32)

    @pl.loop(0, _GROUPS)
    def _(p):
        for b in range(_NBUF):
            k = p * _NBUF + b

            @pl.when((k - _NBUF >= 0) & (k - _NBUF < cnt))
            def _(b=b, k=k):
                pltpu.make_async_copy(vals.at[b], acc.at[cpriv.at[b, 0]],
                                      sems[b]).wait()

            @pl.when(k < _CHUNK_ITERS)
            def _(b=b, k=k):
                pltpu.make_async_copy(rc.at[start + k], ibuf.at[b],
                                      semi[b]).wait()

                @pl.when(cid == 0)
                def _():
                    pltpu.async_copy(
                        w0.at[ibuf.at[b, 0]], vals.at[b], semg[b])

                @pl.when(cid == 1)
                def _():
                    pltpu.async_copy(
                        w1.at[ibuf.at[b, 0]], vals.at[b], semg[b])

        for b in range(_NBUF):
            k = p * _NBUF + b

            @pl.when(k < _CHUNK_ITERS)
            def _(b=b, k=k):
                pltpu.make_async_copy(
                    w0.at[ibuf.at[b, 0]], vals.at[b], semg[b]).wait()

            @pl.when(k < cnt)
            def _(b=b, k=k):
                for m in range(_K // _LANES):
                    iv = ibuf[b, 1, pl.ds(m * _LANES, _LANES)]
                    cpriv[b, 0, pl.ds(m * _LANES, _LANES)] = iv

                    @pl.when(cid == 0)
                    def _():
                        r = jnp.right_shift(iv, 7)
                        c = jnp.bitwise_and(iv, 127)
                        plsc.addupdate_scatter(degtile, [r, c], ones)

                pltpu.async_copy(vals.at[b], acc.at[cpriv.at[b, 0]],
                                 sems[b], add=True)

            @pl.when(k + _NBUF < _CHUNK_ITERS)
            def _(b=b, k=k):
                pltpu.async_copy(rc.at[start + k + _NBUF], ibuf.at[b],
                                 semi[b])

    @pl.when(cnt == _CHUNK_ITERS)
    def _():
        pltpu.make_async_copy(vals.at[0], acc.at[cpriv.at[0, 0]],
                              sems[0]).wait()

    plsc.subcore_barrier()

    @pl.when(cid == 0)
    def _():
        pltpu.sync_copy(degtile, dshared.at[identbuf.at[0]], add=True)

    plsc.subcore_barrier()

    @pl.when(cid == 0)
    def _():
        pltpu.sync_copy(acc.at[sl], s0_out.at[sl])

        @pl.when(sid == _NSUB - 1)
        def _():
            pltpu.sync_copy(acc.at[tl], s0_out.at[tl])

        @pl.when(sid < _DROWS // 8)
        def _():
            dsl = pl.ds(sid * 8, 8)
            pltpu.sync_copy(dshared.at[dsl], deg_out.at[dsl])

    @pl.when(cid == 1)
    def _():
        pltpu.sync_copy(acc.at[sl], s1_out.at[sl])

        @pl.when(sid == _NSUB - 1)
        def _():
            pltpu.sync_copy(acc.at[tl], s1_out.at[tl])


def _sc_segment_sum(w0, w1, rc):
    f32 = jnp.float32
    zval = jnp.zeros((_ROWS_PER_SUB, _HALF), f32)
    zdeg = jnp.zeros((_DROWS, 128), f32)
    ident = jnp.arange(_DROWS, dtype=jnp.int32).reshape(1, _DROWS)
    mesh = plsc.VectorSubcoreMesh(core_axis_name="c", subcore_axis_name="s",
                                  num_cores=2, num_subcores=_NSUB)
    fn = pl.kernel(
        _seg_body,
        out_type=[
            jax.ShapeDtypeStruct((_N, _HALF), f32),
            jax.ShapeDtypeStruct((_N, _HALF), f32),
            jax.ShapeDtypeStruct((_DROWS, 128), f32),
        ],
        mesh=mesh,
        compiler_params=_compiler_params(),
        scratch_types=[
            pltpu.VMEM((_NBUF, 2, _K), jnp.int32),
            pltpu.VMEM((_NBUF, 1, _K), jnp.int32),
            pltpu.VMEM((_NBUF, _K, _HALF), f32),
            pltpu.VMEM((1, _DROWS), jnp.int32),
            pltpu.VMEM((_DROWS, 128), f32),
            pltpu.VMEM_SHARED((_N, _HALF), f32),
            pltpu.VMEM_SHARED((_DROWS, 128), f32),
        ] + [pltpu.SemaphoreType.DMA] * (3 * _NBUF),
    )
    return fn(w0, w1, rc, zval, zdeg, ident)


_BN = 1000


def _dense_x_body(x, w1, b1, ww, wb, ow, ob, u):
    hi = lax.Precision.HIGHEST
    f32 = jnp.float32
    hx = lax.dot_general(x[...], w1[...], (((1,), (1,)), ((), ())),
                         precision=hi, preferred_element_type=f32)
    hx = jnp.maximum(hx + b1[...][None, :], 0.0)
    g1 = lax.dot_general(ww[...][:, :_HID], ow[...], (((0,), (1,)), ((), ())),
                         precision=hi, preferred_element_type=f32)
    cb = lax.dot_general(wb[...][None, :], ow[...], (((1,), (1,)), ((), ())),
                         precision=hi, preferred_element_type=f32)
    y = lax.dot_general(hx, g1, (((1,), (0,)), ((), ())),
                        precision=hi, preferred_element_type=f32)
    u[...] = 2.0 * y + 2.0 * cb + ob[...][None, :]


def _dense_x(X, W1, b1, WW, wb, OW, ob):
    nc = ob.shape[0]
    return pl.pallas_call(
        _dense_x_body,
        grid=(_N // _BN,),
        in_specs=[
            pl.BlockSpec((_BN, _FEAT), lambda i: (i, 0)),
            pl.BlockSpec((_HID, _FEAT), lambda i: (0, 0)),
            pl.BlockSpec((_HID,), lambda i: (0,)),
            pl.BlockSpec((_HID, 2 * _HID), lambda i: (0, 0)),
            pl.BlockSpec((_HID,), lambda i: (0,)),
            pl.BlockSpec((nc, _HID), lambda i: (0, 0)),
            pl.BlockSpec((nc,), lambda i: (0,)),
        ],
        out_specs=pl.BlockSpec((_BN, nc), lambda i: (i, 0)),
        out_shape=jax.ShapeDtypeStruct((_N, nc), jnp.float32),
    )(X, W1, b1, WW, wb, OW, ob)


def _dense_combine_body(u, s0, s1, dg, ww, ow, o):
    hi = lax.Precision.HIGHEST
    f32 = jnp.float32
    scale = 1.0 + 1.0 / jnp.maximum(dg[...], 1.0)
    ha0 = jnp.maximum(s0[...] * scale, 0.0)
    ha1 = jnp.maximum(s1[...] * scale, 0.0)
    g2 = lax.dot_general(ww[...][:, _HID:], ow[...], (((0,), (1,)), ((), ())),
                         precision=hi, preferred_element_type=f32)
    cat = jnp.concatenate([ha0, ha1], axis=1)
    y = lax.dot_general(cat, g2, (((1,), (0,)), ((), ())),
                        precision=hi, preferred_element_type=f32)
    o[...] = u[...] + 2.0 * y


def _dense_combine(U, s0, s1, dg, WW, OW):
    nc = U.shape[1]
    return pl.pallas_call(
        _dense_combine_body,
        grid=(_N // _BN,),
        in_specs=[
            pl.BlockSpec((_BN, nc), lambda i: (i, 0)),
            pl.BlockSpec((_BN, _HALF), lambda i: (i, 0)),
            pl.BlockSpec((_BN, _HALF), lambda i: (i, 0)),
            pl.BlockSpec((_BN, 1), lambda i: (i, 0)),
            pl.BlockSpec((_HID, 2 * _HID), lambda i: (0, 0)),
            pl.BlockSpec((nc, _HID), lambda i: (0, 0)),
        ],
        out_specs=pl.BlockSpec((_BN, nc), lambda i: (i, 0)),
        out_shape=jax.ShapeDtypeStruct((_N, nc), jnp.float32),
    )(U, s0, s1, dg, WW, OW)


def kernel(X, edge_index, MLPX_W, MLPX_b, W_adj, W_W, W_b, out_W, out_b):
    pad = jnp.zeros((_NCHUNK_PAD - _NCHUNK, 2, _K), jnp.int32)
    rc = jnp.concatenate(
        [edge_index.astype(jnp.int32).reshape(2, _NCHUNK, _K).transpose(1, 0, 2),
         pad])
    w0 = W_adj[:, :_HALF]
    w1 = W_adj[:, _HALF:]
    s0, s1, deg80 = _sc_segment_sum(w0, w1, rc)
    u = _dense_x(X, MLPX_W, MLPX_b, W_W, W_b, out_W, out_b)
    dg = deg80.reshape(_DROWS * 128)[:_N].reshape(_N, 1)
    return _dense_combine(u, s0, s1, dg, W_W, out_W)


# speedup vs baseline: 8.2861x; 1.0024x over previous
import dataclasses

import jax
import jax.numpy as jnp
from jax import lax
from jax.experimental import pallas as pl
from jax.experimental.pallas import tpu as pltpu
from jax.experimental.pallas import tpu_sc as plsc

_N = 10000
_E = 320000
_FEAT = 128
_HID = 256
_HALF = _HID // 2
_K = 128
_NSUB = 16
_ROWS_PER_SUB = 624
_TAIL_BASE = _ROWS_PER_SUB * _NSUB
_TAIL_ROWS = _N - _TAIL_BASE
_NCHUNK = _E // _K
_CHUNK_ITERS = -(-_NCHUNK // _NSUB)
_NCHUNK_PAD = _CHUNK_ITERS * _NSUB + _NSUB - 4
_NBUF = 2
_GROUPS = -(-_CHUNK_ITERS // _NBUF)
_DROWS = 80
_LANES = 16


def _compiler_params():
    cp = pltpu.CompilerParams()
    if "needs_layout_passes" in pltpu.CompilerParams.__dataclass_fields__:
        cp = dataclasses.replace(cp, needs_layout_passes=False)
    return cp


def _seg_body(w0, w1, rc, zval, zdeg, ident, s0_out, s1_out, d0_out, d1_out,
              ibuf, vals, identbuf, degtile, acc, dshared,
              semi0, semi1, semg0, semg1):
    cid = lax.axis_index("c")
    sid = lax.axis_index("s")
    base = sid * _ROWS_PER_SUB
    sl = pl.ds(base, _ROWS_PER_SUB)
    tl = pl.ds(_TAIL_BASE, _TAIL_ROWS)
    semi = (semi0, semi1)
    semg = (semg0, semg1)

    start = sid * (_CHUNK_ITERS - 1) + jnp.minimum(sid, 4)
    cnt = jnp.where(sid < 4, _CHUNK_ITERS, _CHUNK_ITERS - 1)

    for b in range(_NBUF):
        pltpu.async_copy(rc.at[start + b], ibuf.at[b], semi[b])
    pltpu.sync_copy(zval.at[pl.ds(0, _ROWS_PER_SUB)], acc.at[sl])

    @pl.when(sid == _NSUB - 1)
    def _():
        pltpu.sync_copy(zval.at[pl.ds(0, _TAIL_ROWS)], acc.at[tl])

    pltpu.sync_copy(zdeg, degtile)
    pltpu.sync_copy(ident, identbuf)

    @pl.when(sid == 0)
    def _():
        pltpu.sync_copy(zdeg, dshared)

    plsc.subcore_barrier()

    ones = jnp.full((_LANES,), 1.0, jnp.float32)

    @pl.loop(0, _GROUPS)
    def _(p):
        for b in range(_NBUF):
            k = p * _NBUF + b

            @pl.when(k < _CHUNK_ITERS)
            def _(b=b, k=k):
                pltpu.make_async_copy(rc.at[start + k], ibuf.at[b],
                                      semi[b]).wait()

                @pl.when(cid == 0)
                def _():
                    pltpu.async_copy(
                        w0.at[ibuf.at[b, 0]], vals.at[b], semg[b])

                @pl.when(cid == 1)
                def _():
                    pltpu.async_copy(
                        w1.at[ibuf.at[b, 0]], vals.at[b], semg[b])

        for b in range(_NBUF):
            k = p * _NBUF + b

            @pl.when(k < _CHUNK_ITERS)
            def _(b=b, k=k):
                pltpu.make_async_copy(
                    w0.at[ibuf.at[b, 0]], vals.at[b], semg[b]).wait()

            @pl.when(k < cnt)
            def _(b=b, k=k):
                pltpu.sync_copy(vals.at[b], acc.at[ibuf.at[b, 1]],
                                add=True)

                @pl.when(cid == b)
                def _():
                    for m in range(_K // _LANES):
                        iv = ibuf[b, 1, pl.ds(m * _LANES, _LANES)]
                        r = jnp.right_shift(iv, 7)
                        c = jnp.bitwise_and(iv, 127)
                        plsc.addupdate_scatter(degtile, [r, c], ones)

            @pl.when(k + _NBUF < _CHUNK_ITERS)
            def _(b=b, k=k):
                pltpu.async_copy(rc.at[start + k + _NBUF], ibuf.at[b],
                                 semi[b])

    plsc.subcore_barrier()

    pltpu.sync_copy(degtile, dshared.at[identbuf.at[0]], add=True)

    plsc.subcore_barrier()

    @pl.when(sid < _DROWS // 8)
    def _():
        dsl = pl.ds(sid * 8, 8)

        @pl.when(cid == 0)
        def _():
            pltpu.sync_copy(dshared.at[dsl], d0_out.at[dsl])

        @pl.when(cid == 1)
        def _():
            pltpu.sync_copy(dshared.at[dsl], d1_out.at[dsl])

    @pl.when(cid == 0)
    def _():
        pltpu.sync_copy(acc.at[sl], s0_out.at[sl])

        @pl.when(sid == _NSUB - 1)
        def _():
            pltpu.sync_copy(acc.at[tl], s0_out.at[tl])

    @pl.when(cid == 1)
    def _():
        pltpu.sync_copy(acc.at[sl], s1_out.at[sl])

        @pl.when(sid == _NSUB - 1)
        def _():
            pltpu.sync_copy(acc.at[tl], s1_out.at[tl])


def _sc_segment_sum(w0, w1, rc):
    f32 = jnp.float32
    zval = jnp.zeros((_ROWS_PER_SUB, _HALF), f32)
    zdeg = jnp.zeros((_DROWS, 128), f32)
    ident = jnp.arange(_DROWS, dtype=jnp.int32).reshape(1, _DROWS)
    mesh = plsc.VectorSubcoreMesh(core_axis_name="c", subcore_axis_name="s",
                                  num_cores=2, num_subcores=_NSUB)
    fn = pl.kernel(
        _seg_body,
        out_type=[
            jax.ShapeDtypeStruct((_N, _HALF), f32),
            jax.ShapeDtypeStruct((_N, _HALF), f32),
            jax.ShapeDtypeStruct((_DROWS, 128), f32),
            jax.ShapeDtypeStruct((_DROWS, 128), f32),
        ],
        mesh=mesh,
        compiler_params=_compiler_params(),
        scratch_types=[
            pltpu.VMEM((_NBUF, 2, _K), jnp.int32),
            pltpu.VMEM((_NBUF, _K, _HALF), f32),
            pltpu.VMEM((1, _DROWS), jnp.int32),
            pltpu.VMEM((_DROWS, 128), f32),
            pltpu.VMEM_SHARED((_N, _HALF), f32),
            pltpu.VMEM_SHARED((_DROWS, 128), f32),
        ] + [pltpu.SemaphoreType.DMA] * (2 * _NBUF),
    )
    return fn(w0, w1, rc, zval, zdeg, ident)


_BN = 1000


def _dense_x_body(x, w1, b1, ww, wb, ow, ob, u):
    hi = lax.Precision.HIGHEST
    f32 = jnp.float32
    hx = lax.dot_general(x[...], w1[...], (((1,), (1,)), ((), ())),
                         precision=hi, preferred_element_type=f32)
    hx = jnp.maximum(hx + b1[...][None, :], 0.0)
    g1 = lax.dot_general(ww[...][:, :_HID], ow[...], (((0,), (1,)), ((), ())),
                         precision=hi, preferred_element_type=f32)
    cb = lax.dot_general(wb[...][None, :], ow[...], (((1,), (1,)), ((), ())),
                         precision=hi, preferred_element_type=f32)
    y = lax.dot_general(hx, g1, (((1,), (0,)), ((), ())),
                        precision=hi, preferred_element_type=f32)
    u[...] = 2.0 * y + 2.0 * cb + ob[...][None, :]


def _dense_x(X, W1, b1, WW, wb, OW, ob):
    nc = ob.shape[0]
    return pl.pallas_call(
        _dense_x_body,
        grid=(_N // _BN,),
        in_specs=[
            pl.BlockSpec((_BN, _FEAT), lambda i: (i, 0)),
            pl.BlockSpec((_HID, _FEAT), lambda i: (0, 0)),
            pl.BlockSpec((_HID,), lambda i: (0,)),
            pl.BlockSpec((_HID, 2 * _HID), lambda i: (0, 0)),
            pl.BlockSpec((_HID,), lambda i: (0,)),
            pl.BlockSpec((nc, _HID), lambda i: (0, 0)),
            pl.BlockSpec((nc,), lambda i: (0,)),
        ],
        out_specs=pl.BlockSpec((_BN, nc), lambda i: (i, 0)),
        out_shape=jax.ShapeDtypeStruct((_N, nc), jnp.float32),
    )(X, W1, b1, WW, wb, OW, ob)


def _dense_combine_body(u, s0, s1, dg0, dg1, ww, ow, o):
    hi = lax.Precision.HIGHEST
    f32 = jnp.float32
    scale = 1.0 + 1.0 / jnp.maximum(dg0[...] + dg1[...], 1.0)
    ha0 = jnp.maximum(s0[...] * scale, 0.0)
    ha1 = jnp.maximum(s1[...] * scale, 0.0)
    g2 = lax.dot_general(ww[...][:, _HID:], ow[...], (((0,), (1,)), ((), ())),
                         precision=hi, preferred_element_type=f32)
    cat = jnp.concatenate([ha0, ha1], axis=1)
    y = lax.dot_general(cat, g2, (((1,), (0,)), ((), ())),
                        precision=hi, preferred_element_type=f32)
    o[...] = u[...] + 2.0 * y


def _dense_combine(U, s0, s1, dg0, dg1, WW, OW):
    nc = U.shape[1]
    return pl.pallas_call(
        _dense_combine_body,
        grid=(_N // _BN,),
        in_specs=[
            pl.BlockSpec((_BN, nc), lambda i: (i, 0)),
            pl.BlockSpec((_BN, _HALF), lambda i: (i, 0)),
            pl.BlockSpec((_BN, _HALF), lambda i: (i, 0)),
            pl.BlockSpec((_BN, 1), lambda i: (i, 0)),
            pl.BlockSpec((_BN, 1), lambda i: (i, 0)),
            pl.BlockSpec((_HID, 2 * _HID), lambda i: (0, 0)),
            pl.BlockSpec((nc, _HID), lambda i: (0, 0)),
        ],
        out_specs=pl.BlockSpec((_BN, nc), lambda i: (i, 0)),
        out_shape=jax.ShapeDtypeStruct((_N, nc), jnp.float32),
    )(U, s0, s1, dg0, dg1, WW, OW)


def kernel(X, edge_index, MLPX_W, MLPX_b, W_adj, W_W, W_b, out_W, out_b):
    pad = jnp.zeros((_NCHUNK_PAD - _NCHUNK, 2, _K), jnp.int32)
    rc = jnp.concatenate(
        [edge_index.astype(jnp.int32).reshape(2, _NCHUNK, _K).transpose(1, 0, 2),
         pad])
    w0 = W_adj[:, :_HALF]
    w1 = W_adj[:, _HALF:]
    s0, s1, d080, d180 = _sc_segment_sum(w0, w1, rc)
    u = _dense_x(X, MLPX_W, MLPX_b, W_W, W_b, out_W, out_b)
    dg0 = d080.reshape(_DROWS * 128)[:_N].reshape(_N, 1)
    dg1 = d180.reshape(_DROWS * 128)[:_N].reshape(_N, 1)
    return _dense_combine(u, s0, s1, dg0, dg1, W_W, out_W)


# speedup vs baseline: 8.3825x; 1.0116x over previous
import dataclasses

import jax
import jax.numpy as jnp
from jax import lax
from jax.experimental import pallas as pl
from jax.experimental.pallas import tpu as pltpu
from jax.experimental.pallas import tpu_sc as plsc

_N = 10000
_E = 320000
_FEAT = 128
_HID = 256
_HALF = _HID // 2
_K = 128
_NSUB = 16
_ROWS_PER_SUB = 624
_TAIL_BASE = _ROWS_PER_SUB * _NSUB
_TAIL_ROWS = _N - _TAIL_BASE
_NCHUNK = _E // _K
_CHUNK_ITERS = -(-_NCHUNK // _NSUB)
_NCHUNK_PAD = _CHUNK_ITERS * _NSUB + _NSUB - 4
_NBUF = 2
_GROUPS = -(-_CHUNK_ITERS // _NBUF)
_DROWS = 80
_LANES = 16


def _compiler_params():
    cp = pltpu.CompilerParams()
    if "needs_layout_passes" in pltpu.CompilerParams.__dataclass_fields__:
        cp = dataclasses.replace(cp, needs_layout_passes=False)
    return cp


def _seg_body(w0, w1, rc, zval, zdeg, ident, s0_out, s1_out, d0_out, d1_out,
              ibuf, vals, identbuf, degtile, acc, dshared,
              semi0, semi1, semg0, semg1):
    cid = lax.axis_index("c")
    sid = lax.axis_index("s")
    base = sid * _ROWS_PER_SUB
    sl = pl.ds(base, _ROWS_PER_SUB)
    tl = pl.ds(_TAIL_BASE, _TAIL_ROWS)
    semi = (semi0, semi1)
    semg = (semg0, semg1)

    start = sid * (_CHUNK_ITERS - 1) + jnp.minimum(sid, 4)
    cnt = jnp.where(sid < 4, _CHUNK_ITERS, _CHUNK_ITERS - 1)

    for b in range(_NBUF):
        pltpu.async_copy(rc.at[start + b], ibuf.at[b], semi[b])
    pltpu.sync_copy(zval.at[pl.ds(0, _ROWS_PER_SUB)], acc.at[sl])

    @pl.when(sid == _NSUB - 1)
    def _():
        pltpu.sync_copy(zval.at[pl.ds(0, _TAIL_ROWS)], acc.at[tl])

    pltpu.sync_copy(zdeg, degtile)
    pltpu.sync_copy(ident, identbuf)

    @pl.when(sid == 0)
    def _():
        pltpu.sync_copy(zdeg, dshared)

    plsc.subcore_barrier()

    ones = jnp.full((_LANES,), 1.0, jnp.float32)

    @pl.loop(0, _GROUPS)
    def _(p):
        for b in range(_NBUF):
            k = p * _NBUF + b

            @pl.when(k < _CHUNK_ITERS)
            def _(b=b, k=k):
                pltpu.make_async_copy(rc.at[start + k], ibuf.at[b],
                                      semi[b]).wait()

                @pl.when(cid == 0)
                def _():
                    pltpu.async_copy(
                        w0.at[ibuf.at[b, 0]], vals.at[b], semg[b])

                @pl.when(cid == 1)
                def _():
                    pltpu.async_copy(
                        w1.at[ibuf.at[b, 0]], vals.at[b], semg[b])

        for b in range(_NBUF):
            k = p * _NBUF + b

            @pl.when(k < _CHUNK_ITERS)
            def _(b=b, k=k):
                pltpu.make_async_copy(
                    w0.at[ibuf.at[b, 0]], vals.at[b], semg[b]).wait()

            @pl.when(k < cnt)
            def _(b=b, k=k):
                pltpu.sync_copy(vals.at[b], acc.at[ibuf.at[b, 1]],
                                add=True)

                @pl.when(cid == b)
                def _():
                    for m in range(_K // _LANES):
                        iv = ibuf[b, 1, pl.ds(m * _LANES, _LANES)]
                        r = jnp.right_shift(iv, 7)
                        c = jnp.bitwise_and(iv, 127)
                        plsc.addupdate_scatter(degtile, [r, c], ones)

            @pl.when(k + _NBUF < _CHUNK_ITERS)
            def _(b=b, k=k):
                pltpu.async_copy(rc.at[start + k + _NBUF], ibuf.at[b],
                                 semi[b])

    plsc.subcore_barrier()

    pltpu.sync_copy(degtile, dshared.at[identbuf.at[0]], add=True)

    plsc.subcore_barrier()

    @pl.when(sid < _DROWS // 8)
    def _():
        dsl = pl.ds(sid * 8, 8)

        @pl.when(cid == 0)
        def _():
            pltpu.sync_copy(dshared.at[dsl], d0_out.at[dsl])

        @pl.when(cid == 1)
        def _():
            pltpu.sync_copy(dshared.at[dsl], d1_out.at[dsl])

    @pl.when(cid == 0)
    def _():
        pltpu.sync_copy(acc.at[sl], s0_out.at[sl])

        @pl.when(sid == _NSUB - 1)
        def _():
            pltpu.sync_copy(acc.at[tl], s0_out.at[tl])

    @pl.when(cid == 1)
    def _():
        pltpu.sync_copy(acc.at[sl], s1_out.at[sl])

        @pl.when(sid == _NSUB - 1)
        def _():
            pltpu.sync_copy(acc.at[tl], s1_out.at[tl])


def _sc_segment_sum(w0, w1, rc):
    f32 = jnp.float32
    zval = jnp.zeros((_ROWS_PER_SUB, _HALF), f32)
    zdeg = jnp.zeros((_DROWS, 128), f32)
    ident = jnp.arange(_DROWS, dtype=jnp.int32).reshape(1, _DROWS)
    mesh = plsc.VectorSubcoreMesh(core_axis_name="c", subcore_axis_name="s",
                                  num_cores=2, num_subcores=_NSUB)
    fn = pl.kernel(
        _seg_body,
        out_type=[
            jax.ShapeDtypeStruct((_N, _HALF), f32),
            jax.ShapeDtypeStruct((_N, _HALF), f32),
            jax.ShapeDtypeStruct((_DROWS, 128), f32),
            jax.ShapeDtypeStruct((_DROWS, 128), f32),
        ],
        mesh=mesh,
        compiler_params=_compiler_params(),
        scratch_types=[
            pltpu.VMEM((_NBUF, 2, _K), jnp.int32),
            pltpu.VMEM((_NBUF, _K, _HALF), f32),
            pltpu.VMEM((1, _DROWS), jnp.int32),
            pltpu.VMEM((_DROWS, 128), f32),
            pltpu.VMEM_SHARED((_N, _HALF), f32),
            pltpu.VMEM_SHARED((_DROWS, 128), f32),
        ] + [pltpu.SemaphoreType.DMA] * (2 * _NBUF),
    )
    return fn(w0, w1, rc, zval, zdeg, ident)


_BN = 2000


def _dense_x_body(x, w1, b1, ww, wb, ow, ob, u):
    hi = lax.Precision.HIGHEST
    f32 = jnp.float32
    hx = lax.dot_general(x[...], w1[...], (((1,), (1,)), ((), ())),
                         precision=hi, preferred_element_type=f32)
    hx = jnp.maximum(hx + b1[...][None, :], 0.0)
    g1 = lax.dot_general(ww[...][:, :_HID], ow[...], (((0,), (1,)), ((), ())),
                         precision=hi, preferred_element_type=f32)
    cb = lax.dot_general(wb[...][None, :], ow[...], (((1,), (1,)), ((), ())),
                         precision=hi, preferred_element_type=f32)
    y = lax.dot_general(hx, g1, (((1,), (0,)), ((), ())),
                        precision=hi, preferred_element_type=f32)
    u[...] = 2.0 * y + 2.0 * cb + ob[...][None, :]


def _dense_x(X, W1, b1, WW, wb, OW, ob):
    nc = ob.shape[0]
    return pl.pallas_call(
        _dense_x_body,
        grid=(_N // _BN,),
        in_specs=[
            pl.BlockSpec((_BN, _FEAT), lambda i: (i, 0)),
            pl.BlockSpec((_HID, _FEAT), lambda i: (0, 0)),
            pl.BlockSpec((_HID,), lambda i: (0,)),
            pl.BlockSpec((_HID, 2 * _HID), lambda i: (0, 0)),
            pl.BlockSpec((_HID,), lambda i: (0,)),
            pl.BlockSpec((nc, _HID), lambda i: (0, 0)),
            pl.BlockSpec((nc,), lambda i: (0,)),
        ],
        out_specs=pl.BlockSpec((_BN, nc), lambda i: (i, 0)),
        out_shape=jax.ShapeDtypeStruct((_N, nc), jnp.float32),
    )(X, W1, b1, WW, wb, OW, ob)


def _dense_combine_body(u, s0, s1, dg0, dg1, ww, ow, o):
    hi = lax.Precision.HIGHEST
    f32 = jnp.float32
    scale = 1.0 + 1.0 / jnp.maximum(dg0[...] + dg1[...], 1.0)
    ha0 = jnp.maximum(s0[...] * scale, 0.0)
    ha1 = jnp.maximum(s1[...] * scale, 0.0)
    g2 = lax.dot_general(ww[...][:, _HID:], ow[...], (((0,), (1,)), ((), ())),
                         precision=hi, preferred_element_type=f32)
    cat = jnp.concatenate([ha0, ha1], axis=1)
    y = lax.dot_general(cat, g2, (((1,), (0,)), ((), ())),
                        precision=hi, preferred_element_type=f32)
    o[...] = u[...] + 2.0 * y


def _dense_combine(U, s0, s1, dg0, dg1, WW, OW):
    nc = U.shape[1]
    return pl.pallas_call(
        _dense_combine_body,
        grid=(_N // _BN,),
        in_specs=[
            pl.BlockSpec((_BN, nc), lambda i: (i, 0)),
            pl.BlockSpec((_BN, _HALF), lambda i: (i, 0)),
            pl.BlockSpec((_BN, _HALF), lambda i: (i, 0)),
            pl.BlockSpec((_BN, 1), lambda i: (i, 0)),
            pl.BlockSpec((_BN, 1), lambda i: (i, 0)),
            pl.BlockSpec((_HID, 2 * _HID), lambda i: (0, 0)),
            pl.BlockSpec((nc, _HID), lambda i: (0, 0)),
        ],
        out_specs=pl.BlockSpec((_BN, nc), lambda i: (i, 0)),
        out_shape=jax.ShapeDtypeStruct((_N, nc), jnp.float32),
    )(U, s0, s1, dg0, dg1, WW, OW)


def kernel(X, edge_index, MLPX_W, MLPX_b, W_adj, W_W, W_b, out_W, out_b):
    pad = jnp.zeros((_NCHUNK_PAD - _NCHUNK, 2, _K), jnp.int32)
    rc = jnp.concatenate(
        [edge_index.astype(jnp.int32).reshape(2, _NCHUNK, _K).transpose(1, 0, 2),
         pad])
    w0 = W_adj[:, :_HALF]
    w1 = W_adj[:, _HALF:]
    s0, s1, d080, d180 = _sc_segment_sum(w0, w1, rc)
    u = _dense_x(X, MLPX_W, MLPX_b, W_W, W_b, out_W, out_b)
    dg0 = d080.reshape(_DROWS * 128)[:_N].reshape(_N, 1)
    dg1 = d180.reshape(_DROWS * 128)[:_N].reshape(_N, 1)
    return _dense_combine(u, s0, s1, dg0, dg1, W_W, out_W)
